# Initial kernel scaffold; baseline (speedup 1.0000x reference)
import jax
import jax.numpy as jnp
from jax.experimental import pallas as pl


def kernel(fmaps, rois):
    raise NotImplementedError("write your pallas kernel here")


# speedup vs baseline: 12.1372x; 12.1372x over previous
import jax
import jax.numpy as jnp
from jax import lax
from jax.experimental import pallas as pl
from jax.experimental.pallas import tpu as pltpu

O_H, O_W = 7, 7
MAXBIN = 10

NEG = float("-inf")


def _bin_bounds(lo, delta, i, n):
    s_raw = lo + (i * delta + (n - 1)) // n
    e = lo + ((i + 1) * delta + (n - 1)) // n
    s = jnp.where(s_raw == e, s_raw - 1, s_raw)
    return s, e


def _tc_body(rois_ref, fmap_ref, out_ref, rm_ref):
    g = pl.program_id(0)
    b = g // 64
    r = g % 64
    H, W = 64, 64
    x1 = (rois_ref[b, r, 0] * W).astype(jnp.int32)
    y1 = (rois_ref[b, r, 1] * H).astype(jnp.int32)
    x2 = (rois_ref[b, r, 2] * W).astype(jnp.int32)
    y2 = (rois_ref[b, r, 3] * H).astype(jnp.int32)
    dx = x2 + 1 - x1
    dy = y2 + 1 - y1

    rms = []
    for oy in range(O_H):
        ys, ye = _bin_bounds(y1, dy, oy, O_H)
        h = ye - ys
        s0 = jnp.clip(ys, 0, H - MAXBIN)
        off = ys - s0
        window = fmap_ref[0, pl.ds(s0, MAXBIN), :, :]
        i = lax.broadcasted_iota(jnp.int32, (MAXBIN, 1, 1), 0)
        m = (i >= off) & (i < off + h)
        rms.append(jnp.max(jnp.where(m, window, NEG), axis=0))
    rm_ref[...] = jnp.stack(rms, axis=1)

    cols = []
    for ox in range(O_W):
        xs, xe = _bin_bounds(x1, dx, ox, O_W)
        w = xe - xs
        s0 = jnp.clip(xs, 0, W - MAXBIN)
        off = xs - s0
        window = rm_ref[pl.ds(s0, MAXBIN), :, :]
        i = lax.broadcasted_iota(jnp.int32, (MAXBIN, 1, 1), 0)
        m = (i >= off) & (i < off + w)
        cols.append(jnp.max(jnp.where(m, window, NEG), axis=0))
    out_ref[0, 0] = jnp.stack(cols, axis=1)


def kernel(fmaps, rois):
    B, H, W, C = fmaps.shape
    R = rois.shape[1]
    return pl.pallas_call(
        _tc_body,
        grid=(B * R,),
        in_specs=[
            pl.BlockSpec(memory_space=pltpu.SMEM),
            pl.BlockSpec((1, H, W, C), lambda g: (g // 64, 0, 0, 0)),
        ],
        out_specs=pl.BlockSpec(
            (1, 1, O_H, O_W, C), lambda g: (g // 64, g % 64, 0, 0, 0)
        ),
        out_shape=jax.ShapeDtypeStruct((B, R, O_H, O_W, C), jnp.float32),
        scratch_shapes=[pltpu.VMEM((W, O_H, C), jnp.float32)],
    )(rois, fmaps)
